# baseline (speedup 1.0000x reference)
import functools
import math

import jax
import jax.numpy as jnp
import numpy as np
from jax import lax
from jax.experimental import pallas as pl
from jax.experimental.pallas import tpu as pltpu

B, S, T = 4, 256, 256
D_MODEL, NHEAD, D_FF, N_EXPERTS = 256, 4, 1024, 8
DH = D_MODEL // NHEAD
SRC_V, TGT_V = 32000, 32000
MAX_LEN = 261
N_TOK = B * S


def _pe_np(max_len, d):
    pos = np.arange(max_len, dtype=np.float32)[:, None]
    div = np.exp(np.arange(0, d, 2, dtype=np.float32) * (-math.log(10000.0) / d))
    pe = np.zeros((max_len, d), dtype=np.float32)
    pe[:, 0::2] = np.sin(pos * div)
    pe[:, 1::2] = np.cos(pos * div)
    return pe


_PE = _pe_np(MAX_LEN, D_MODEL)


def _dotT(a, b):
    return lax.dot_general(a, b, (((1,), (1,)), ((), ())),
                           preferred_element_type=jnp.float32)


def _dot(a, b):
    return lax.dot_general(a, b, (((1,), (0,)), ((), ())),
                           preferred_element_type=jnp.float32)


def _rowsum(y):
    ones = jnp.ones((y.shape[1], 128), jnp.float32)
    return _dot(y, ones)[:, 0:1]


def _ln(y, g, b):
    d = y.shape[1]
    m = _rowsum(y) * (1.0 / d)
    v = _rowsum(y * y) * (1.0 / d) - m * m
    return (y - m) * lax.rsqrt(v + 1e-5) * g + b


def _mha_block(x, kv, wq, bq, wk, bk, wv, bv, wo, bo, causal):
    q = _dotT(x, wq) + bq
    k = _dotT(kv, wk) + bk
    v = _dotT(kv, wv) + bv
    scale = 1.0 / math.sqrt(DH)
    if causal:
        ri = lax.broadcasted_iota(jnp.int32, (S, S), 0)
        ci = lax.broadcasted_iota(jnp.int32, (S, S), 1)
        mask = jnp.where(ci > ri, jnp.float32(-1e30), jnp.float32(0.0))
    rows_out = []
    for b in range(B):
        r0 = b * S
        houts = []
        for h in range(NHEAD):
            c0 = h * DH
            qh = q[r0:r0 + S, c0:c0 + DH]
            kh = k[r0:r0 + S, c0:c0 + DH]
            vh = v[r0:r0 + S, c0:c0 + DH]
            s = _dotT(qh, kh) * scale
            if causal:
                s = s + mask
            s = s - jnp.max(s, axis=-1, keepdims=True)
            p = jnp.exp(s)
            p = p * (1.0 / _rowsum(p))
            houts.append(_dot(p, vh))
        rows_out.append(jnp.concatenate(houts, axis=1))
    o = jnp.concatenate(rows_out, axis=0)
    return _dotT(o, wo) + bo


def _top2_comb(x, rw, rb):
    glog = _dotT(x, rw) + rb
    glog = glog - jnp.max(glog, axis=-1, keepdims=True)
    gexp = jnp.exp(glog)
    gate = gexp / jnp.sum(gexp, axis=-1, keepdims=True)
    eidx = lax.broadcasted_iota(jnp.int32, gate.shape, 1)
    m1 = jnp.max(gate, axis=-1, keepdims=True)
    i1 = jnp.min(jnp.where(gate == m1, eidx, N_EXPERTS), axis=-1, keepdims=True)
    one1 = eidx == i1
    gate2 = jnp.where(one1, jnp.float32(-1.0), gate)
    m2 = jnp.max(gate2, axis=-1, keepdims=True)
    i2 = jnp.min(jnp.where(gate2 == m2, eidx, N_EXPERTS), axis=-1, keepdims=True)
    comb = jnp.where(one1 | (eidx == i2), gate, jnp.float32(0.0))
    return gate, comb, eidx


_EPB = 2

def _layer_body(is_dec, n_ln2, *refs):
    n_in = 1 + (1 if is_dec else 0) + 8 + 2 + (10 if is_dec else 0) \
        + 6 + 2 + 2 * n_ln2
    ins, outs, scr = refs[:n_in], refs[n_in:n_in + 2], refs[n_in + 2:]
    it = iter(ins)
    x_ref = next(it)
    mem_ref = next(it) if is_dec else None
    attn_w = [next(it) for _ in range(8)]
    ln1g, ln1b = next(it), next(it)
    if is_dec:
        xattn_w = [next(it) for _ in range(8)]
        ln2g, ln2b = next(it), next(it)
    rw_ref, rb_ref = next(it), next(it)
    w1_ref, b1_ref, w2_ref, b2_ref = next(it), next(it), next(it), next(it)
    lng, lnb = next(it), next(it)
    ln_extra = [next(it) for _ in range(2 * n_ln2)]
    out_ref, aux_ref = outs
    xa_ref, comb_ref, xab_ref = scr

    e = pl.program_id(0)

    @pl.when(e == 0)
    def _prologue():
        x = x_ref[...]
        a = _mha_block(x, x, *[w[...] for w in attn_w], causal=is_dec)
        xa = _ln(x + a, ln1g[...], ln1b[...])
        if is_dec:
            c = _mha_block(xa, mem_ref[...], *[w[...] for w in xattn_w],
                           causal=False)
            xa = _ln(xa + c, ln2g[...], ln2b[...])
        gate, comb, _ = _top2_comb(xa, rw_ref[...], rb_ref[...])
        ep = jnp.mean(gate, axis=0, keepdims=True)
        aux_ref[...] = jnp.sum(ep * jnp.log(ep + 1e-9), keepdims=True
                               ).reshape(1, 1)
        xa_ref[...] = xa
        xab_ref[...] = xa.astype(jnp.bfloat16)
        comb_ref[...] = comb
        out_ref[...] = xa

    xab = xab_ref[...]
    eidx = lax.broadcasted_iota(jnp.int32, (N_TOK, N_EXPERTS), 1)
    comb = comb_ref[...]
    acc = out_ref[...]
    for j in range(_EPB):
        eid = e * _EPB + j
        col = jnp.sum(jnp.where(eidx == eid, comb, 0.0), axis=-1,
                      keepdims=True)
        h = _dotT(xab, w1_ref[j].astype(jnp.bfloat16))
        h = jnp.maximum(h + b1_ref[j], 0.0).astype(jnp.bfloat16)
        acc = acc + col * _dotT(h, w2_ref[j].astype(jnp.bfloat16))
    out_ref[...] = acc

    @pl.when(e == N_EXPERTS // _EPB - 1)
    def _fin():
        t = out_ref[...] + _dot(comb_ref[...], b2_ref[...])
        t = _ln(t, lng[...], lnb[...])
        if n_ln2:
            t = _ln(t, ln_extra[0][...], ln_extra[1][...])
        out_ref[...] = t


def _layer(x, lp, mem=None, g2=None, b2=None):
    is_dec = mem is not None
    n_ln2 = 1 if g2 is not None else 0
    row = lambda a: a.reshape(1, -1)
    full = lambda shp: pl.BlockSpec(shp, lambda e: (0,) * len(shp))

    def attn_ins(p):
        return [p['Wq'], row(p['bq']), p['Wk'], row(p['bk']),
                p['Wv'], row(p['bv']), p['Wo'], row(p['bo'])]

    def attn_specs():
        return [full((D_MODEL, D_MODEL)), full((1, D_MODEL))] * 4

    moe = lp['moe']
    ins = [x]
    in_specs = [full((N_TOK, D_MODEL))]
    if is_dec:
        ins += [mem]
        in_specs += [full((N_TOK, D_MODEL))]
    ins += attn_ins(lp['attn']) + [row(lp['ln1_g']), row(lp['ln1_b'])]
    in_specs += attn_specs() + [full((1, D_MODEL))] * 2
    if is_dec:
        ins += attn_ins(lp['xattn']) + [row(lp['ln2_g']), row(lp['ln2_b'])]
        in_specs += attn_specs() + [full((1, D_MODEL))] * 2
    post_g = lp['ln3_g'] if is_dec else lp['ln2_g']
    post_b = lp['ln3_b'] if is_dec else lp['ln2_b']
    ins += [moe['rW'], row(moe['rb']), moe['W1'],
            moe['b1'].reshape(N_EXPERTS, 1, D_FF), moe['W2'],
            moe['b2'],
            row(post_g), row(post_b)]
    in_specs += [
        full((N_EXPERTS, D_MODEL)), full((1, N_EXPERTS)),
        pl.BlockSpec((_EPB, D_FF, D_MODEL), lambda e: (e, 0, 0)),
        pl.BlockSpec((_EPB, 1, D_FF), lambda e: (e, 0, 0)),
        pl.BlockSpec((_EPB, D_MODEL, D_FF), lambda e: (e, 0, 0)),
        full((N_EXPERTS, D_MODEL)),
        full((1, D_MODEL)), full((1, D_MODEL)),
    ]
    if n_ln2:
        ins += [row(g2), row(b2)]
        in_specs += [full((1, D_MODEL))] * 2

    out, aux = pl.pallas_call(
        functools.partial(_layer_body, is_dec, n_ln2),
        grid=(N_EXPERTS // _EPB,),
        in_specs=in_specs,
        out_specs=[full((N_TOK, D_MODEL)), full((1, 1))],
        out_shape=[jax.ShapeDtypeStruct((N_TOK, D_MODEL), jnp.float32),
                   jax.ShapeDtypeStruct((1, 1), jnp.float32)],
        scratch_shapes=[pltpu.VMEM((N_TOK, D_MODEL), jnp.float32),
                        pltpu.VMEM((N_TOK, N_EXPERTS), jnp.float32),
                        pltpu.VMEM((N_TOK, D_MODEL), jnp.bfloat16)],
    )(*ins)
    return out, aux


_VBLK = 3200


def _logits_body(x_ref, w_ref, b_ref, o_ref, xb_ref):
    @pl.when(pl.program_id(0) == 0)
    def _cast():
        xb_ref[...] = x_ref[...].astype(jnp.bfloat16)

    o_ref[...] = _dotT(xb_ref[...],
                       w_ref[...].astype(jnp.bfloat16)) + b_ref[...]


def _logits(x, w, b):
    n = x.shape[0]
    v = w.shape[0]
    return pl.pallas_call(
        _logits_body,
        grid=(v // _VBLK,),
        in_specs=[
            pl.BlockSpec((n, D_MODEL), lambda i: (0, 0)),
            pl.BlockSpec((_VBLK, D_MODEL), lambda i: (i, 0)),
            pl.BlockSpec((1, _VBLK), lambda i: (0, i)),
        ],
        out_specs=pl.BlockSpec((n, _VBLK), lambda i: (0, i)),
        out_shape=jax.ShapeDtypeStruct((n, v), jnp.float32),
        scratch_shapes=[pltpu.VMEM((n, D_MODEL), jnp.bfloat16)],
    )(x, w, b.reshape(1, -1))


def kernel(params, src_ids, tgt_ids_in):
    pe = jnp.asarray(_PE)
    scale = math.sqrt(D_MODEL)
    src = params['src_emb'][src_ids] * scale + pe[:src_ids.shape[1]]
    tgt = params['tgt_emb'][tgt_ids_in] * scale + pe[:tgt_ids_in.shape[1]]

    auxes = []
    mem = src.reshape(N_TOK, D_MODEL)
    n_enc = len(params['enc'])
    for li, lp in enumerate(params['enc']):
        last = li == n_enc - 1
        mem, aux = _layer(mem, lp,
                          g2=params['ln_enc_g'] if last else None,
                          b2=params['ln_enc_b'] if last else None)
        auxes.append(aux)

    out = tgt.reshape(N_TOK, D_MODEL)
    n_dec = len(params['dec'])
    for li, lp in enumerate(params['dec']):
        last = li == n_dec - 1
        out, aux = _layer(out, lp, mem=mem,
                          g2=params['ln_dec_g'] if last else None,
                          b2=params['ln_dec_b'] if last else None)
        auxes.append(aux)

    logits = _logits(out, params['out_W'],
                     params['out_b']).reshape(B, T, TGT_V)
    total_aux = sum(auxes)[0, 0]
    return logits, total_aux


# speedup vs baseline: 1.1054x; 1.1054x over previous
import functools
import math

import jax
import jax.numpy as jnp
import numpy as np
from jax import lax
from jax.experimental import pallas as pl
from jax.experimental.pallas import tpu as pltpu
from jax.experimental.pallas import tpu_sc as plsc

B, S, T = 4, 256, 256
D_MODEL, NHEAD, D_FF, N_EXPERTS = 256, 4, 1024, 8
DH = D_MODEL // NHEAD
SRC_V, TGT_V = 32000, 32000
MAX_LEN = 261
N_TOK = B * S


def _pe_np(max_len, d):
    pos = np.arange(max_len, dtype=np.float32)[:, None]
    div = np.exp(np.arange(0, d, 2, dtype=np.float32) * (-math.log(10000.0) / d))
    pe = np.zeros((max_len, d), dtype=np.float32)
    pe[:, 0::2] = np.sin(pos * div)
    pe[:, 1::2] = np.cos(pos * div)
    return pe


_PE = _pe_np(MAX_LEN, D_MODEL)


def _dotT(a, b):
    return lax.dot_general(a, b, (((1,), (1,)), ((), ())),
                           preferred_element_type=jnp.float32)


def _dot(a, b):
    return lax.dot_general(a, b, (((1,), (0,)), ((), ())),
                           preferred_element_type=jnp.float32)


def _ln(y, g, b):
    m = jnp.mean(y, axis=-1, keepdims=True)
    v = jnp.mean((y - m) ** 2, axis=-1, keepdims=True)
    return (y - m) * lax.rsqrt(v + 1e-5) * g + b


def _mha_block(x, kv, wq, bq, wk, bk, wv, bv, wo, bo, causal):
    q = _dotT(x, wq) + bq
    k = _dotT(kv, wk) + bk
    v = _dotT(kv, wv) + bv
    scale = 1.0 / math.sqrt(DH)
    if causal:
        ri = lax.broadcasted_iota(jnp.int32, (S, S), 0)
        ci = lax.broadcasted_iota(jnp.int32, (S, S), 1)
        mask = jnp.where(ci > ri, jnp.float32(-1e30), jnp.float32(0.0))
    rows_out = []
    for b in range(B):
        r0 = b * S
        houts = []
        for h in range(NHEAD):
            c0 = h * DH
            qh = q[r0:r0 + S, c0:c0 + DH]
            kh = k[r0:r0 + S, c0:c0 + DH]
            vh = v[r0:r0 + S, c0:c0 + DH]
            s = _dotT(qh, kh) * scale
            if causal:
                s = s + mask
            s = s - jnp.max(s, axis=-1, keepdims=True)
            p = jnp.exp(s)
            p = p / jnp.sum(p, axis=-1, keepdims=True)
            houts.append(_dot(p, vh))
        rows_out.append(jnp.concatenate(houts, axis=1))
    o = jnp.concatenate(rows_out, axis=0)
    return _dotT(o, wo) + bo


def _top2_comb(x, rw, rb):
    glog = _dotT(x, rw) + rb
    glog = glog - jnp.max(glog, axis=-1, keepdims=True)
    gexp = jnp.exp(glog)
    gate = gexp / jnp.sum(gexp, axis=-1, keepdims=True)
    eidx = lax.broadcasted_iota(jnp.int32, gate.shape, 1)
    m1 = jnp.max(gate, axis=-1, keepdims=True)
    i1 = jnp.min(jnp.where(gate == m1, eidx, N_EXPERTS), axis=-1, keepdims=True)
    one1 = eidx == i1
    gate2 = jnp.where(one1, jnp.float32(-1.0), gate)
    m2 = jnp.max(gate2, axis=-1, keepdims=True)
    i2 = jnp.min(jnp.where(gate2 == m2, eidx, N_EXPERTS), axis=-1, keepdims=True)
    comb = jnp.where(one1 | (eidx == i2), gate, jnp.float32(0.0))
    return gate, comb, eidx


def _sc_gather(table, ids):
    info = plsc.get_sparse_core_info()
    nw = info.num_cores * info.num_subcores
    n = ids.shape[0]
    b_per_w = n // nw
    mesh = plsc.VectorSubcoreMesh(core_axis_name="c", subcore_axis_name="s")

    @functools.partial(
        pl.kernel, mesh=mesh,
        out_type=jax.ShapeDtypeStruct((n, D_MODEL), jnp.float32),
        scratch_types=[
            pltpu.VMEM((b_per_w,), jnp.int32),
            pltpu.VMEM((b_per_w, D_MODEL), jnp.float32),
            pltpu.SemaphoreType.DMA,
        ],
    )
    def gk(table_hbm, idx_hbm, out_hbm, idx_v, rows_v, sem):
        wid = lax.axis_index("s") * info.num_cores + lax.axis_index("c")
        base = wid * b_per_w
        pltpu.sync_copy(idx_hbm.at[pl.ds(base, b_per_w)], idx_v)
        pltpu.async_copy(table_hbm.at[idx_v], rows_v, sem).wait()
        pltpu.sync_copy(rows_v, out_hbm.at[pl.ds(base, b_per_w)])

    return gk(table, ids)


_EPB = 2

def _layer_body(is_dec, n_ln2, *refs):
    n_in = 1 + (1 if is_dec else 0) + 8 + 2 + (10 if is_dec else 0) \
        + 6 + 2 + 2 * n_ln2
    ins, outs, scr = refs[:n_in], refs[n_in:n_in + 2], refs[n_in + 2:]
    it = iter(ins)
    x_ref = next(it)
    mem_ref = next(it) if is_dec else None
    attn_w = [next(it) for _ in range(8)]
    ln1g, ln1b = next(it), next(it)
    if is_dec:
        xattn_w = [next(it) for _ in range(8)]
        ln2g, ln2b = next(it), next(it)
    rw_ref, rb_ref = next(it), next(it)
    w1_ref, b1_ref, w2_ref, b2_ref = next(it), next(it), next(it), next(it)
    lng, lnb = next(it), next(it)
    ln_extra = [next(it) for _ in range(2 * n_ln2)]
    out_ref, aux_ref = outs
    xa_ref, comb_ref, xab_ref = scr

    e = pl.program_id(0)

    @pl.when(e == 0)
    def _prologue():
        x = x_ref[...]
        a = _mha_block(x, x, *[w[...] for w in attn_w], causal=is_dec)
        xa = _ln(x + a, ln1g[...], ln1b[...])
        if is_dec:
            c = _mha_block(xa, mem_ref[...], *[w[...] for w in xattn_w],
                           causal=False)
            xa = _ln(xa + c, ln2g[...], ln2b[...])
        gate, comb, _ = _top2_comb(xa, rw_ref[...], rb_ref[...])
        ep = jnp.mean(gate, axis=0, keepdims=True)
        aux_ref[...] = jnp.sum(ep * jnp.log(ep + 1e-9), keepdims=True
                               ).reshape(1, 1)
        xa_ref[...] = xa
        xab_ref[...] = xa.astype(jnp.bfloat16)
        comb_ref[...] = comb
        out_ref[...] = xa

    xab = xab_ref[...]
    eidx = lax.broadcasted_iota(jnp.int32, (N_TOK, N_EXPERTS), 1)
    comb = comb_ref[...]
    acc = out_ref[...]
    for j in range(_EPB):
        eid = e * _EPB + j
        col = jnp.sum(jnp.where(eidx == eid, comb, 0.0), axis=-1,
                      keepdims=True)
        h = _dotT(xab, w1_ref[j].astype(jnp.bfloat16))
        h = jnp.maximum(h + b1_ref[j], 0.0).astype(jnp.bfloat16)
        acc = acc + col * _dotT(h, w2_ref[j].astype(jnp.bfloat16))
    out_ref[...] = acc

    @pl.when(e == N_EXPERTS // _EPB - 1)
    def _fin():
        t = out_ref[...] + _dot(comb_ref[...], b2_ref[...])
        t = _ln(t, lng[...], lnb[...])
        if n_ln2:
            t = _ln(t, ln_extra[0][...], ln_extra[1][...])
        out_ref[...] = t


def _layer(x, lp, mem=None, g2=None, b2=None):
    is_dec = mem is not None
    n_ln2 = 1 if g2 is not None else 0
    row = lambda a: a.reshape(1, -1)
    full = lambda shp: pl.BlockSpec(shp, lambda e: (0,) * len(shp))

    def attn_ins(p):
        return [p['Wq'], row(p['bq']), p['Wk'], row(p['bk']),
                p['Wv'], row(p['bv']), p['Wo'], row(p['bo'])]

    def attn_specs():
        return [full((D_MODEL, D_MODEL)), full((1, D_MODEL))] * 4

    moe = lp['moe']
    ins = [x]
    in_specs = [full((N_TOK, D_MODEL))]
    if is_dec:
        ins += [mem]
        in_specs += [full((N_TOK, D_MODEL))]
    ins += attn_ins(lp['attn']) + [row(lp['ln1_g']), row(lp['ln1_b'])]
    in_specs += attn_specs() + [full((1, D_MODEL))] * 2
    if is_dec:
        ins += attn_ins(lp['xattn']) + [row(lp['ln2_g']), row(lp['ln2_b'])]
        in_specs += attn_specs() + [full((1, D_MODEL))] * 2
    post_g = lp['ln3_g'] if is_dec else lp['ln2_g']
    post_b = lp['ln3_b'] if is_dec else lp['ln2_b']
    ins += [moe['rW'], row(moe['rb']), moe['W1'],
            moe['b1'].reshape(N_EXPERTS, 1, D_FF), moe['W2'],
            moe['b2'],
            row(post_g), row(post_b)]
    in_specs += [
        full((N_EXPERTS, D_MODEL)), full((1, N_EXPERTS)),
        pl.BlockSpec((_EPB, D_FF, D_MODEL), lambda e: (e, 0, 0)),
        pl.BlockSpec((_EPB, 1, D_FF), lambda e: (e, 0, 0)),
        pl.BlockSpec((_EPB, D_MODEL, D_FF), lambda e: (e, 0, 0)),
        full((N_EXPERTS, D_MODEL)),
        full((1, D_MODEL)), full((1, D_MODEL)),
    ]
    if n_ln2:
        ins += [row(g2), row(b2)]
        in_specs += [full((1, D_MODEL))] * 2

    out, aux = pl.pallas_call(
        functools.partial(_layer_body, is_dec, n_ln2),
        grid=(N_EXPERTS // _EPB,),
        in_specs=in_specs,
        out_specs=[full((N_TOK, D_MODEL)), full((1, 1))],
        out_shape=[jax.ShapeDtypeStruct((N_TOK, D_MODEL), jnp.float32),
                   jax.ShapeDtypeStruct((1, 1), jnp.float32)],
        scratch_shapes=[pltpu.VMEM((N_TOK, D_MODEL), jnp.float32),
                        pltpu.VMEM((N_TOK, N_EXPERTS), jnp.float32),
                        pltpu.VMEM((N_TOK, D_MODEL), jnp.bfloat16)],
    )(*ins)
    return out, aux


_VBLK = 3200


def _logits_body(x_ref, w_ref, b_ref, o_ref, xb_ref):
    @pl.when(pl.program_id(0) == 0)
    def _cast():
        xb_ref[...] = x_ref[...].astype(jnp.bfloat16)

    o_ref[...] = _dotT(xb_ref[...],
                       w_ref[...].astype(jnp.bfloat16)) + b_ref[...]


def _logits(x, w, b):
    n = x.shape[0]
    v = w.shape[0]
    return pl.pallas_call(
        _logits_body,
        grid=(v // _VBLK,),
        in_specs=[
            pl.BlockSpec((n, D_MODEL), lambda i: (0, 0)),
            pl.BlockSpec((_VBLK, D_MODEL), lambda i: (i, 0)),
            pl.BlockSpec((1, _VBLK), lambda i: (0, i)),
        ],
        out_specs=pl.BlockSpec((n, _VBLK), lambda i: (0, i)),
        out_shape=jax.ShapeDtypeStruct((n, v), jnp.float32),
        scratch_shapes=[pltpu.VMEM((n, D_MODEL), jnp.bfloat16)],
    )(x, w, b.reshape(1, -1))


def kernel(params, src_ids, tgt_ids_in):
    scale = math.sqrt(D_MODEL)
    pe_src = jnp.asarray(np.tile(_PE[:S], (B, 1)))
    pe_tgt = jnp.asarray(np.tile(_PE[:T], (B, 1)))
    src = _sc_gather(params['src_emb'],
                     src_ids.reshape(-1).astype(jnp.int32)) * scale + pe_src
    tgt = _sc_gather(params['tgt_emb'],
                     tgt_ids_in.reshape(-1).astype(jnp.int32)) * scale + pe_tgt

    auxes = []
    mem = src
    n_enc = len(params['enc'])
    for li, lp in enumerate(params['enc']):
        last = li == n_enc - 1
        mem, aux = _layer(mem, lp,
                          g2=params['ln_enc_g'] if last else None,
                          b2=params['ln_enc_b'] if last else None)
        auxes.append(aux)

    out = tgt
    n_dec = len(params['dec'])
    for li, lp in enumerate(params['dec']):
        last = li == n_dec - 1
        out, aux = _layer(out, lp, mem=mem,
                          g2=params['ln_dec_g'] if last else None,
                          b2=params['ln_dec_b'] if last else None)
        auxes.append(aux)

    logits = _logits(out, params['out_W'],
                     params['out_b']).reshape(B, T, TGT_V)
    total_aux = sum(auxes)[0, 0]
    return logits, total_aux
